# baseline (speedup 1.0000x reference)
import functools

import jax
import jax.numpy as jnp
from jax import lax
from jax.experimental import pallas as pl
from jax.experimental.pallas import tpu as pltpu
from jax.experimental.pallas import tpu_sc as plsc

_N = 26
_VOCAB = 100000
_DIM = 16

_mesh = plsc.VectorSubcoreMesh(
    core_axis_name="c", subcore_axis_name="s", num_cores=1, num_subcores=1
)


@functools.partial(
    pl.kernel,
    mesh=_mesh,
    out_type=jax.ShapeDtypeStruct((1, _N * _DIM), jnp.float32),
    scratch_types=[
        pltpu.VMEM((32,), jnp.int32),
        pltpu.VMEM((_N, _DIM, 128), jnp.float32),
        pltpu.VMEM((_N * _DIM,), jnp.float32),
        pltpu.SemaphoreType.DMA,
    ],
    compiler_params=pltpu.CompilerParams(
        use_tc_tiling_on_sc=True,
        needs_layout_passes=False,
        skip_device_barrier=True,
    ),
)
def _gather(idx_hbm, table_hbm, out_hbm, idx_v, bufs, rows, sem):
    pltpu.sync_copy(idx_hbm, idx_v.at[pl.ds(0, _N)])
    v0 = idx_v[pl.ds(0, 16)]
    v1 = idx_v[pl.ds(16, 16)]
    scalars = [v0[f] for f in range(16)] + [v1[f] for f in range(_N - 16)]
    copies = []
    for f in range(_N):
        s = scalars[f]
        blk = (s // 128) * 128
        copies.append(
            pltpu.async_copy(
                table_hbm.at[f].at[:, pl.ds(blk, 128)], bufs.at[f], sem
            )
        )
    iota = lax.iota(jnp.int32, 16)
    for f in range(_N):
        copies[f].wait()
        lane = jnp.full((16,), scalars[f] % 128, jnp.int32)
        row = plsc.load_gather(bufs.at[f], [iota, lane])
        rows[pl.ds(_DIM * f, _DIM)] = row
    pltpu.sync_copy(rows, out_hbm.at[0])


def kernel(tables, indices):
    return _gather(indices.astype(jnp.int32), tables.transpose(0, 2, 1))


# speedup vs baseline: 1.1278x; 1.1278x over previous
import functools

import jax
import jax.numpy as jnp
from jax import lax
from jax.experimental import pallas as pl
from jax.experimental.pallas import tpu as pltpu
from jax.experimental.pallas import tpu_sc as plsc

_N = 26
_VOCAB = 100000
_DIM = 16

_mesh = plsc.VectorSubcoreMesh(
    core_axis_name="c", subcore_axis_name="s", num_cores=1, num_subcores=16
)


@functools.partial(
    pl.kernel,
    mesh=_mesh,
    out_type=jax.ShapeDtypeStruct((1, _N * _DIM), jnp.float32),
    scratch_types=[
        pltpu.VMEM((32,), jnp.int32),
        pltpu.VMEM((2, _DIM, 128), jnp.float32),
        pltpu.VMEM((2 * _DIM,), jnp.float32),
        pltpu.SemaphoreType.DMA,
        pltpu.SemaphoreType.DMA,
    ],
    compiler_params=pltpu.CompilerParams(
        use_tc_tiling_on_sc=True, needs_layout_passes=False
    ),
)
def _gather(idx_hbm, table_hbm, out_hbm, idx_v, bufs, rowbuf, sem, osem):
    sid = lax.axis_index("s")
    pltpu.sync_copy(idx_hbm, idx_v.at[pl.ds(0, _N)])
    iota = lax.iota(jnp.int32, 16)

    def field_scalar(fvec):
        return plsc.load_gather(idx_v, [fvec])[0]

    s0 = field_scalar(jnp.full((16,), sid, jnp.int32))
    two = sid < (_N - 16)
    f1 = jnp.where(two, sid + 16, sid)
    s1 = field_scalar(jnp.full((16,), f1, jnp.int32))

    c0 = pltpu.async_copy(
        table_hbm.at[sid].at[:, pl.ds((s0 // 128) * 128, 128)], bufs.at[0], sem
    )
    c1 = pltpu.async_copy(
        table_hbm.at[f1].at[:, pl.ds((s1 // 128) * 128, 128)], bufs.at[1], sem
    )
    c0.wait()
    rowbuf[pl.ds(0, _DIM)] = plsc.load_gather(
        bufs.at[0], [iota, jnp.full((16,), s0 % 128, jnp.int32)]
    )
    o0 = pltpu.async_copy(
        rowbuf.at[pl.ds(0, _DIM)],
        out_hbm.at[0].at[pl.ds(pl.multiple_of(_DIM * sid, _DIM), _DIM)],
        osem,
    )
    c1.wait()
    rowbuf[pl.ds(_DIM, _DIM)] = plsc.load_gather(
        bufs.at[1], [iota, jnp.full((16,), s1 % 128, jnp.int32)]
    )
    o0.wait()

    @pl.when(two)
    def _():
        pltpu.async_copy(
            rowbuf.at[pl.ds(_DIM, _DIM)],
            out_hbm.at[0].at[pl.ds(pl.multiple_of(_DIM * f1, _DIM), _DIM)],
            osem,
        ).wait()


def kernel(tables, indices):
    return _gather(indices.astype(jnp.int32), tables.transpose(0, 2, 1))
